# Initial kernel scaffold; baseline (speedup 1.0000x reference)
import jax
import jax.numpy as jnp
from jax.experimental import pallas as pl


def kernel(h, edge_index0, edge_index1, W0, al0, ar0, b0, W1, al1, ar1, b1, sem_W1, sem_b1, sem_W2):
    raise NotImplementedError("write your pallas kernel here")


# speedup vs baseline: 35.7649x; 35.7649x over previous
import functools

import jax
import jax.numpy as jnp
from jax import lax
from jax.experimental import pallas as pl
from jax.experimental.pallas import tpu as pltpu
from jax.experimental.pallas import tpu_sc as plsc

NC = 2
NS = 16
L = 16
C = 80


def _mm_body(h_ref, w0_ref, al0_ref, ar0_ref, w1_ref, al1_ref, ar1_ref,
             f0_ref, f1_ref, el0_ref, er0_ref, el1_ref, er1_ref):
    hb = h_ref[...]
    f0 = jnp.dot(hb, w0_ref[...], preferred_element_type=jnp.float32,
                 precision=lax.Precision.HIGHEST)
    f1 = jnp.dot(hb, w1_ref[...], preferred_element_type=jnp.float32,
                 precision=lax.Precision.HIGHEST)
    f0_ref[...] = f0
    f1_ref[...] = f1
    el0_ref[...] = jnp.dot(f0, al0_ref[...], preferred_element_type=jnp.float32,
                           precision=lax.Precision.HIGHEST)
    er0_ref[...] = jnp.dot(f0, ar0_ref[...], preferred_element_type=jnp.float32,
                           precision=lax.Precision.HIGHEST)
    el1_ref[...] = jnp.dot(f1, al1_ref[...], preferred_element_type=jnp.float32,
                           precision=lax.Precision.HIGHEST)
    er1_ref[...] = jnp.dot(f1, ar1_ref[...], preferred_element_type=jnp.float32,
                           precision=lax.Precision.HIGHEST)


def _mm_call(h, W0, al0, ar0, W1, al1, ar1):
    n, d = h.shape
    rb = 2000
    grid = (n // rb,)
    full = lambda s: pl.BlockSpec(s, lambda i: (0, 0))
    return pl.pallas_call(
        _mm_body,
        grid=grid,
        in_specs=[
            pl.BlockSpec((rb, d), lambda i: (i, 0)),
            full((d, d)), full((d, 1)), full((d, 1)),
            full((d, d)), full((d, 1)), full((d, 1)),
        ],
        out_specs=[
            pl.BlockSpec((rb, d), lambda i: (i, 0)),
            pl.BlockSpec((rb, d), lambda i: (i, 0)),
            pl.BlockSpec((rb, 1), lambda i: (i, 0)),
            pl.BlockSpec((rb, 1), lambda i: (i, 0)),
            pl.BlockSpec((rb, 1), lambda i: (i, 0)),
            pl.BlockSpec((rb, 1), lambda i: (i, 0)),
        ],
        out_shape=[
            jax.ShapeDtypeStruct((n, d), jnp.float32),
            jax.ShapeDtypeStruct((n, d), jnp.float32),
            jax.ShapeDtypeStruct((n, 1), jnp.float32),
            jax.ShapeDtypeStruct((n, 1), jnp.float32),
            jax.ShapeDtypeStruct((n, 1), jnp.float32),
            jax.ShapeDtypeStruct((n, 1), jnp.float32),
        ],
    )(h, W0, al0.reshape(d, 1), ar0.reshape(d, 1),
      W1, al1.reshape(d, 1), ar1.reshape(d, 1))


def _sca_call(src0, dst0, src1, dst1, el0, er0, el1, er1):
    n = el0.shape[0]
    e = src0.shape[0]
    epw = e // (NC * NS)
    chunks = epw // C
    NB = 4
    body_steps = chunks // NB * NB

    def body(src0_h, dst0_h, src1_h, dst1_h, el0_h, er0_h, el1_h, er1_h,
             ee_h, s_h, el_v, er_v, sv, dv, ee_all, s_local, *sems):
        cid = lax.axis_index("c")
        sid = lax.axis_index("s")
        wid = cid * NS + sid
        srcs = (src0_h, src1_h)
        dsts = (dst0_h, dst1_h)
        els = (el0_h, el1_h)
        ers = (er0_h, er1_h)

        for p in range(2):
            pltpu.sync_copy(els[p], el_v)
            pltpu.sync_copy(ers[p], er_v)

            def szero(i, _):
                s_local[pl.ds(i * L, L)] = jnp.zeros((L,), jnp.float32)
                return 0
            lax.fori_loop(0, n // L, szero, 0)

            base = wid * epw

            def issue_idx(k, b):
                pltpu.async_copy(srcs[p].at[pl.ds(base + k * C, C)],
                                 sv.at[b], sems[b])
                pltpu.async_copy(dsts[p].at[pl.ds(base + k * C, C)],
                                 dv.at[b], sems[b])

            def process(k, b):
                pltpu.make_async_copy(srcs[p].at[pl.ds(0, C)], sv.at[b],
                                      sems[b]).wait()
                pltpu.make_async_copy(dsts[p].at[pl.ds(0, C)], dv.at[b],
                                      sems[b]).wait()
                for i in range(C // L):
                    sidx = sv[b, pl.ds(i * L, L)]
                    didx = dv[b, pl.ds(i * L, L)]
                    x = plsc.load_gather(el_v, [sidx]) + plsc.load_gather(er_v, [didx])
                    ee = jnp.exp(jnp.where(x > 0.0, x, x * 0.2))
                    ee_all[pl.ds(k * C + i * L, L)] = ee
                    plsc.addupdate_scatter(s_local, [didx], ee)

            for b in range(NB):
                issue_idx(b, b)

            def quad_body(k4, _):
                for t in range(NB):
                    k = NB * k4 + t
                    process(k, t)

                    @pl.when(k + NB < chunks)
                    def _():
                        issue_idx(k + NB, t)
                return 0
            lax.fori_loop(0, body_steps // NB, quad_body, 0)
            for k in range(body_steps, chunks):
                process(k, k % NB)

            pltpu.sync_copy(ee_all, ee_h.at[pl.ds(p * e + base, epw)])
            pltpu.sync_copy(s_local, s_h.at[(cid * 2 + p) * NS + sid])

    kern = pl.kernel(
        body,
        out_type=[
            jax.ShapeDtypeStruct((2 * e,), jnp.float32),
            jax.ShapeDtypeStruct((2 * NC * NS, n), jnp.float32),
        ],
        mesh=plsc.VectorSubcoreMesh(core_axis_name="c", subcore_axis_name="s",
                                    num_cores=NC, num_subcores=NS),
        compiler_params=pltpu.CompilerParams(needs_layout_passes=False),
        scratch_types=[
            pltpu.VMEM((n,), jnp.float32),
            pltpu.VMEM((n,), jnp.float32),
            pltpu.VMEM((4, C), jnp.int32),
            pltpu.VMEM((4, C), jnp.int32),
            pltpu.VMEM((epw,), jnp.float32),
            pltpu.VMEM((n,), jnp.float32),
        ] + [pltpu.SemaphoreType.DMA] * 4,
    )
    return kern(src0, dst0, src1, dst1, el0, er0, el1, er1)


def _scb_call(src0, dst0, src1, dst1, ee, feat0, feat1):
    n = feat0.shape[0]
    d = feat0.shape[1]
    e = src0.shape[0]
    epw = e // (NC * NS)
    chunks = epw // C
    NB = 3
    body_steps = (chunks - 2) // NB * NB
    srows = n // NS // 8 * 8
    lrows = n - srows * (NS - 1)
    zr = 16

    def body(src0_h, dst0_h, src1_h, dst1_h, ee_h, f0_h, f1_h, msg_h,
             sv, dv, ev, rg0, rg1, rg2, msg_acc, *sems):
        semi = sems[:3]
        semg = sems[3:]
        cid = lax.axis_index("c")
        sid = lax.axis_index("s")
        wid = cid * NS + sid
        srcs = (src0_h, src1_h)
        dsts = (dst0_h, dst1_h)
        feats = (f0_h, f1_h)
        rgs = (rg0, rg1, rg2)

        def zfill(r, _):
            for j in range(d // L):
                rg0[r, pl.ds(j * L, L)] = jnp.zeros((L,), jnp.float32)
            return 0

        for p in range(2):
            lax.fori_loop(0, zr, zfill, 0)
            for kz in range(srows // zr):
                pltpu.sync_copy(rg0.at[pl.ds(0, zr), :],
                                msg_acc.at[pl.ds(sid * srows + kz * zr, zr), :])

            @pl.when(sid == NS - 1)
            def _():
                for kz in range(srows // zr, lrows // zr):
                    pltpu.sync_copy(
                        rg0.at[pl.ds(0, zr), :],
                        msg_acc.at[pl.ds((NS - 1) * srows + kz * zr, zr), :])
            plsc.subcore_barrier()

            base = wid * epw

            def issue_idx(k, b):
                pltpu.async_copy(srcs[p].at[pl.ds(base + k * C, C)],
                                 sv.at[b], semi[b])
                pltpu.async_copy(dsts[p].at[pl.ds(base + k * C, C)],
                                 dv.at[b], semi[b])
                pltpu.async_copy(ee_h.at[pl.ds(p * e + base + k * C, C)],
                                 ev.at[b], semi[b])

            def wait_idx(b):
                pltpu.make_async_copy(srcs[p].at[pl.ds(0, C)], sv.at[b],
                                      semi[b]).wait()
                pltpu.make_async_copy(dsts[p].at[pl.ds(0, C)], dv.at[b],
                                      semi[b]).wait()
                pltpu.make_async_copy(ee_h.at[pl.ds(0, C)], ev.at[b],
                                      semi[b]).wait()

            def issue_gather(b):
                pltpu.async_copy(feats[p].at[sv.at[b]], rgs[b], semg[b])

            def scale_scatter(b):
                pltpu.make_async_copy(feats[p].at[sv.at[b]], rgs[b],
                                      semg[b]).wait()

                def scale_body(g, _):
                    ee16 = ev[b, pl.ds(g * L, L)]
                    for l in range(L):
                        q = g * L + l
                        sc = ee16[l]
                        for j in range(d // L):
                            rgs[b][q, pl.ds(j * L, L)] = (
                                rgs[b][q, pl.ds(j * L, L)] * sc)
                    return 0
                lax.fori_loop(0, C // L, scale_body, 0)
                pltpu.sync_copy(rgs[b], msg_acc.at[dv.at[b]], add=True)

            for b in range(NB):
                issue_idx(b, b)
            wait_idx(0)
            issue_gather(0)
            wait_idx(1)
            issue_gather(1)

            def tri_body(k3, _):
                for t in range(NB):
                    k = NB * k3 + t
                    scale_scatter(t)
                    issue_idx_b = t

                    @pl.when(k + NB < chunks)
                    def _():
                        issue_idx(k + NB, issue_idx_b)
                    g_b = (t + 2) % NB
                    wait_idx(g_b)
                    issue_gather(g_b)
                return 0
            lax.fori_loop(0, body_steps // NB, tri_body, 0)
            for k in range(body_steps, chunks):
                b = k % NB
                scale_scatter(b)

                @pl.when(k + NB < chunks)
                def _():
                    issue_idx(k + NB, b)
                if k + 2 < chunks:
                    g_b = (b + 2) % NB
                    wait_idx(g_b)
                    issue_gather(g_b)

            plsc.subcore_barrier()

            @pl.when(sid < NS - 1)
            def _():
                pltpu.sync_copy(
                    msg_acc.at[pl.ds(sid * srows, srows), :],
                    msg_h.at[cid * 2 + p, pl.ds(sid * srows, srows), :])

            @pl.when(sid == NS - 1)
            def _():
                pltpu.sync_copy(
                    msg_acc.at[pl.ds((NS - 1) * srows, lrows), :],
                    msg_h.at[cid * 2 + p, pl.ds((NS - 1) * srows, lrows), :])

    kern = pl.kernel(
        body,
        out_type=jax.ShapeDtypeStruct((2 * NC, n, d), jnp.float32),
        mesh=plsc.VectorSubcoreMesh(core_axis_name="c", subcore_axis_name="s",
                                    num_cores=NC, num_subcores=NS),
        compiler_params=pltpu.CompilerParams(needs_layout_passes=False),
        scratch_types=[
            pltpu.VMEM((3, C), jnp.int32),
            pltpu.VMEM((3, C), jnp.int32),
            pltpu.VMEM((3, C), jnp.float32),
            pltpu.VMEM((C, d), jnp.float32),
            pltpu.VMEM((C, d), jnp.float32),
            pltpu.VMEM((C, d), jnp.float32),
            pltpu.VMEM_SHARED((n, d), jnp.float32),
        ] + [pltpu.SemaphoreType.DMA] * 6,
    )
    return kern(src0, dst0, src1, dst1, ee, feat0, feat1)


def _k2_body(mp_ref, sp_ref, b0_ref, b1_ref, w1_ref, b1s_ref, w2_ref,
             z0_ref, z1_ref, tw_ref):
    mp = mp_ref[...]
    sp = sp_ref[...]
    m0 = mp[0] + mp[2]
    m1 = mp[1] + mp[3]
    s0 = (jnp.sum(sp[:, 0:16], axis=1, keepdims=True)
          + jnp.sum(sp[:, 32:48], axis=1, keepdims=True))
    s1 = (jnp.sum(sp[:, 16:32], axis=1, keepdims=True)
          + jnp.sum(sp[:, 48:64], axis=1, keepdims=True))
    d = 128

    def path(m, s, b_ref):
        x = m / (s + 1e-9) + b_ref[...]
        return jnp.where(x > 0.0, x, jnp.exp(x) - 1.0)

    z0 = path(m0, s0, b0_ref)
    z1 = path(m1, s1, b1_ref)
    z0_ref[...] = z0
    z1_ref[...] = z1

    def sem(z):
        t = jnp.tanh(jnp.dot(z, w1_ref[...], preferred_element_type=jnp.float32,
                             precision=lax.Precision.HIGHEST) + b1s_ref[...])
        return jnp.sum(jnp.dot(t, w2_ref[...], preferred_element_type=jnp.float32,
                               precision=lax.Precision.HIGHEST))

    col = lax.broadcasted_iota(jnp.int32, (1, 1, d), 2)
    tw_ref[...] = (jnp.where(col == 0, sem(z0), 0.0)
                   + jnp.where(col == 1, sem(z1), 0.0))


def _k2_call(msgp, s2d, b0, b1, sem_W1, sem_b1, sem_W2):
    n = msgp.shape[1]
    d = 128
    hid = sem_W1.shape[1]
    rb = 1000
    grid = (n // rb,)
    full = lambda s: pl.BlockSpec(s, lambda i: (0, 0))
    return pl.pallas_call(
        _k2_body,
        grid=grid,
        in_specs=[
            pl.BlockSpec((4, rb, d), lambda i: (0, i, 0)),
            pl.BlockSpec((rb, 64), lambda i: (i, 0)),
            full((1, d)), full((1, d)),
            full((d, hid)), full((1, hid)), full((hid, 1)),
        ],
        out_specs=[
            pl.BlockSpec((rb, d), lambda i: (i, 0)),
            pl.BlockSpec((rb, d), lambda i: (i, 0)),
            pl.BlockSpec((1, 1, d), lambda i: (i, 0, 0)),
        ],
        out_shape=[
            jax.ShapeDtypeStruct((n, d), jnp.float32),
            jax.ShapeDtypeStruct((n, d), jnp.float32),
            jax.ShapeDtypeStruct((grid[0], 1, d), jnp.float32),
        ],
    )(msgp, s2d, b0.reshape(1, d), b1.reshape(1, d),
      sem_W1, sem_b1.reshape(1, hid), sem_W2)


def _k3_body(n, tw_ref, z0_ref, z1_ref, o_ref):
    tws = jnp.sum(tw_ref[...], axis=0) * (1.0 / n)
    w0 = tws[:, 0:1]
    w1 = tws[:, 1:2]
    m = jnp.maximum(w0, w1)
    e0 = jnp.exp(w0 - m)
    e1 = jnp.exp(w1 - m)
    o_ref[...] = (e0 * z0_ref[...] + e1 * z1_ref[...]) / (e0 + e1)


def _k3_call(tw, z0, z1):
    n, d = z0.shape
    rb = 1000
    grid = (n // rb,)
    return pl.pallas_call(
        functools.partial(_k3_body, n),
        grid=grid,
        in_specs=[
            pl.BlockSpec(tw.shape, lambda i: (0, 0, 0)),
            pl.BlockSpec((rb, d), lambda i: (i, 0)),
            pl.BlockSpec((rb, d), lambda i: (i, 0)),
        ],
        out_specs=pl.BlockSpec((rb, d), lambda i: (i, 0)),
        out_shape=jax.ShapeDtypeStruct((n, d), jnp.float32),
    )(tw, z0, z1)


def kernel(h, edge_index0, edge_index1, W0, al0, ar0, b0, W1, al1, ar1, b1,
           sem_W1, sem_b1, sem_W2):
    n, d = h.shape
    f0, f1, el0, er0, el1, er1 = _mm_call(h, W0, al0, ar0, W1, al1, ar1)
    s0a, d0a = edge_index0[0], edge_index0[1]
    s1a, d1a = edge_index1[0], edge_index1[1]
    ee, s_part = _sca_call(s0a, d0a, s1a, d1a,
                           el0.reshape(n), er0.reshape(n),
                           el1.reshape(n), er1.reshape(n))
    msgp = _scb_call(s0a, d0a, s1a, d1a, ee, f0, f1)
    z0, z1, tw = _k2_call(msgp, s_part.T, b0, b1, sem_W1, sem_b1, sem_W2)
    return _k3_call(tw, z0, z1)


# speedup vs baseline: 36.5766x; 1.0227x over previous
import functools

import jax
import jax.numpy as jnp
from jax import lax
from jax.experimental import pallas as pl
from jax.experimental.pallas import tpu as pltpu
from jax.experimental.pallas import tpu_sc as plsc

NC = 2
NS = 16
L = 16
C = 80


def _mm_body(h_ref, w0_ref, al0_ref, ar0_ref, w1_ref, al1_ref, ar1_ref,
             f0_ref, f1_ref, el0_ref, er0_ref, el1_ref, er1_ref):
    hb = h_ref[...]
    f0 = jnp.dot(hb, w0_ref[...], preferred_element_type=jnp.float32,
                 precision=lax.Precision.HIGHEST)
    f1 = jnp.dot(hb, w1_ref[...], preferred_element_type=jnp.float32,
                 precision=lax.Precision.HIGHEST)
    f0_ref[...] = f0
    f1_ref[...] = f1
    el0_ref[...] = jnp.dot(f0, al0_ref[...], preferred_element_type=jnp.float32,
                           precision=lax.Precision.HIGHEST)
    er0_ref[...] = jnp.dot(f0, ar0_ref[...], preferred_element_type=jnp.float32,
                           precision=lax.Precision.HIGHEST)
    el1_ref[...] = jnp.dot(f1, al1_ref[...], preferred_element_type=jnp.float32,
                           precision=lax.Precision.HIGHEST)
    er1_ref[...] = jnp.dot(f1, ar1_ref[...], preferred_element_type=jnp.float32,
                           precision=lax.Precision.HIGHEST)


def _mm_call(h, W0, al0, ar0, W1, al1, ar1):
    n, d = h.shape
    rb = 2000
    grid = (n // rb,)
    full = lambda s: pl.BlockSpec(s, lambda i: (0, 0))
    return pl.pallas_call(
        _mm_body,
        grid=grid,
        in_specs=[
            pl.BlockSpec((rb, d), lambda i: (i, 0)),
            full((d, d)), full((d, 1)), full((d, 1)),
            full((d, d)), full((d, 1)), full((d, 1)),
        ],
        out_specs=[
            pl.BlockSpec((rb, d), lambda i: (i, 0)),
            pl.BlockSpec((rb, d), lambda i: (i, 0)),
            pl.BlockSpec((rb, 1), lambda i: (i, 0)),
            pl.BlockSpec((rb, 1), lambda i: (i, 0)),
            pl.BlockSpec((rb, 1), lambda i: (i, 0)),
            pl.BlockSpec((rb, 1), lambda i: (i, 0)),
        ],
        out_shape=[
            jax.ShapeDtypeStruct((n, d), jnp.float32),
            jax.ShapeDtypeStruct((n, d), jnp.float32),
            jax.ShapeDtypeStruct((n, 1), jnp.float32),
            jax.ShapeDtypeStruct((n, 1), jnp.float32),
            jax.ShapeDtypeStruct((n, 1), jnp.float32),
            jax.ShapeDtypeStruct((n, 1), jnp.float32),
        ],
    )(h, W0, al0.reshape(d, 1), ar0.reshape(d, 1),
      W1, al1.reshape(d, 1), ar1.reshape(d, 1))


def _sca_call(src0, dst0, src1, dst1, el0, er0, el1, er1):
    n = el0.shape[0]
    e = src0.shape[0]
    epw = e // (NC * NS)
    chunks = epw // C
    NB = 4
    body_steps = chunks // NB * NB

    def body(src0_h, dst0_h, src1_h, dst1_h, el0_h, er0_h, el1_h, er1_h,
             ee_h, s_h, el_v, er_v, sv, dv, ee_all, s_local, *sems):
        cid = lax.axis_index("c")
        sid = lax.axis_index("s")
        wid = cid * NS + sid
        srcs = (src0_h, src1_h)
        dsts = (dst0_h, dst1_h)
        els = (el0_h, el1_h)
        ers = (er0_h, er1_h)

        for p in range(2):
            pltpu.sync_copy(els[p], el_v)
            pltpu.sync_copy(ers[p], er_v)

            def szero(i, _):
                s_local[pl.ds(i * L, L)] = jnp.zeros((L,), jnp.float32)
                return 0
            lax.fori_loop(0, n // L, szero, 0)

            base = wid * epw

            def issue_idx(k, b):
                pltpu.async_copy(srcs[p].at[pl.ds(base + k * C, C)],
                                 sv.at[b], sems[b])
                pltpu.async_copy(dsts[p].at[pl.ds(base + k * C, C)],
                                 dv.at[b], sems[b])

            def process(k, b):
                pltpu.make_async_copy(srcs[p].at[pl.ds(0, C)], sv.at[b],
                                      sems[b]).wait()
                pltpu.make_async_copy(dsts[p].at[pl.ds(0, C)], dv.at[b],
                                      sems[b]).wait()
                for i in range(C // L):
                    sidx = sv[b, pl.ds(i * L, L)]
                    didx = dv[b, pl.ds(i * L, L)]
                    x = plsc.load_gather(el_v, [sidx]) + plsc.load_gather(er_v, [didx])
                    ee = jnp.exp(jnp.where(x > 0.0, x, x * 0.2))
                    ee_all[pl.ds(k * C + i * L, L)] = ee
                    plsc.addupdate_scatter(s_local, [didx], ee)

            for b in range(NB):
                issue_idx(b, b)

            def quad_body(k4, _):
                for t in range(NB):
                    k = NB * k4 + t
                    process(k, t)

                    @pl.when(k + NB < chunks)
                    def _():
                        issue_idx(k + NB, t)
                return 0
            lax.fori_loop(0, body_steps // NB, quad_body, 0)
            for k in range(body_steps, chunks):
                process(k, k % NB)

            pltpu.sync_copy(ee_all, ee_h.at[pl.ds(p * e + base, epw)])
            pltpu.sync_copy(s_local, s_h.at[(cid * 2 + p) * NS + sid])

    kern = pl.kernel(
        body,
        out_type=[
            jax.ShapeDtypeStruct((2 * e,), jnp.float32),
            jax.ShapeDtypeStruct((2 * NC * NS, n), jnp.float32),
        ],
        mesh=plsc.VectorSubcoreMesh(core_axis_name="c", subcore_axis_name="s",
                                    num_cores=NC, num_subcores=NS),
        compiler_params=pltpu.CompilerParams(needs_layout_passes=False),
        scratch_types=[
            pltpu.VMEM((n,), jnp.float32),
            pltpu.VMEM((n,), jnp.float32),
            pltpu.VMEM((4, C), jnp.int32),
            pltpu.VMEM((4, C), jnp.int32),
            pltpu.VMEM((epw,), jnp.float32),
            pltpu.VMEM((n,), jnp.float32),
        ] + [pltpu.SemaphoreType.DMA] * 4,
    )
    return kern(src0, dst0, src1, dst1, el0, er0, el1, er1)


def _scb_call(src0, dst0, src1, dst1, ee, feat0, feat1):
    n = feat0.shape[0]
    d = feat0.shape[1]
    e = src0.shape[0]
    epw = e // (NC * NS)
    chunks = epw // C
    NG = 4
    NI = 6
    srows = n // NS // 8 * 8
    lrows = n - srows * (NS - 1)
    zr = 16

    def body(src0_h, dst0_h, src1_h, dst1_h, ee_h, f0_h, f1_h, msg_h,
             sv, dv, ev, rgbuf, msg_acc, semi, semg):
        cid = lax.axis_index("c")
        sid = lax.axis_index("s")
        wid = cid * NS + sid
        srcs = (src0_h, src1_h)
        dsts = (dst0_h, dst1_h)
        feats = (f0_h, f1_h)

        def zfill(r, _):
            for j in range(d // L):
                rgbuf[r, pl.ds(j * L, L)] = jnp.zeros((L,), jnp.float32)
            return 0

        for p in range(2):
            lax.fori_loop(0, zr, zfill, 0)
            def zcopy(kz, _):
                pltpu.sync_copy(rgbuf.at[pl.ds(0, zr), :],
                                msg_acc.at[pl.ds(sid * srows + kz * zr, zr), :])
                return 0
            lax.fori_loop(0, srows // zr, zcopy, 0)

            @pl.when(sid == NS - 1)
            def _():
                def zcopy2(kz, _):
                    pltpu.sync_copy(
                        rgbuf.at[pl.ds(0, zr), :],
                        msg_acc.at[pl.ds((NS - 1) * srows + kz * zr, zr), :])
                    return 0
                lax.fori_loop(srows // zr, lrows // zr, zcopy2, 0)
            plsc.subcore_barrier()

            base = wid * epw

            def issue_idx(k, bi):
                pltpu.async_copy(srcs[p].at[pl.ds(base + k * C, C)],
                                 sv.at[bi], semi.at[bi])
                pltpu.async_copy(dsts[p].at[pl.ds(base + k * C, C)],
                                 dv.at[bi], semi.at[bi])
                pltpu.async_copy(ee_h.at[pl.ds(p * e + base + k * C, C)],
                                 ev.at[bi], semi.at[bi])

            def wait_idx(bi):
                pltpu.make_async_copy(srcs[p].at[pl.ds(0, C)], sv.at[bi],
                                      semi.at[bi]).wait()
                pltpu.make_async_copy(dsts[p].at[pl.ds(0, C)], dv.at[bi],
                                      semi.at[bi]).wait()
                pltpu.make_async_copy(ee_h.at[pl.ds(0, C)], ev.at[bi],
                                      semi.at[bi]).wait()

            def rg(bg):
                return rgbuf.at[pl.ds(bg * C, C), :]

            def issue_gather(bg, bi):
                pltpu.async_copy(feats[p].at[sv.at[bi]], rg(bg), semg.at[bg])

            def wait_gather(bg, bi):
                pltpu.make_async_copy(feats[p].at[sv.at[bi]], rg(bg),
                                      semg.at[bg]).wait()

            def scale(bg, bi):
                def scale_body(g, _):
                    ee16 = ev[bi, pl.ds(g * L, L)]
                    for l in range(L):
                        q = g * L + l
                        sc = ee16[l]
                        for j in range(d // L):
                            rgbuf[bg * C + q, pl.ds(j * L, L)] = (
                                rgbuf[bg * C + q, pl.ds(j * L, L)] * sc)
                    return 0
                lax.fori_loop(0, C // L, scale_body, 0)

            def issue_scatter(bg, bi):
                pltpu.async_copy(rg(bg), msg_acc.at[dv.at[bi]], semg.at[bg],
                                 add=True)

            def wait_scatter(bg, bi):
                pltpu.make_async_copy(rg(bg), msg_acc.at[dv.at[bi]],
                                      semg.at[bg]).wait()

            for j in range(4):
                issue_idx(j, j)
            wait_idx(0)
            issue_gather(0, 0)
            wait_idx(1)
            issue_gather(1, 1)

            def step_body(k, _):
                bg = lax.rem(k, NG)
                bi = lax.rem(k, NI)
                wait_gather(bg, bi)
                scale(bg, bi)
                issue_scatter(bg, bi)

                @pl.when(k >= 2)
                def _():
                    wait_scatter(lax.rem(k + 2, NG), lax.rem(k + 4, NI))

                @pl.when(k + 4 < chunks)
                def _():
                    issue_idx(k + 4, lax.rem(k + 4, NI))

                @pl.when(k + 2 < chunks)
                def _():
                    wait_idx(lax.rem(k + 2, NI))
                    issue_gather(lax.rem(k + 2, NG), lax.rem(k + 2, NI))
                return 0
            lax.fori_loop(0, chunks, step_body, 0)
            wait_scatter((chunks - 2) % NG, (chunks - 2) % NI)
            wait_scatter((chunks - 1) % NG, (chunks - 1) % NI)

            plsc.subcore_barrier()

            @pl.when(sid < NS - 1)
            def _():
                pltpu.sync_copy(
                    msg_acc.at[pl.ds(sid * srows, srows), :],
                    msg_h.at[cid * 2 + p, pl.ds(sid * srows, srows), :])

            @pl.when(sid == NS - 1)
            def _():
                pltpu.sync_copy(
                    msg_acc.at[pl.ds((NS - 1) * srows, lrows), :],
                    msg_h.at[cid * 2 + p, pl.ds((NS - 1) * srows, lrows), :])

    kern = pl.kernel(
        body,
        out_type=jax.ShapeDtypeStruct((2 * NC, n, d), jnp.float32),
        mesh=plsc.VectorSubcoreMesh(core_axis_name="c", subcore_axis_name="s",
                                    num_cores=NC, num_subcores=NS),
        compiler_params=pltpu.CompilerParams(needs_layout_passes=False),
        scratch_types=[
            pltpu.VMEM((6, C), jnp.int32),
            pltpu.VMEM((6, C), jnp.int32),
            pltpu.VMEM((6, C), jnp.float32),
            pltpu.VMEM((4 * C, d), jnp.float32),
            pltpu.VMEM_SHARED((n, d), jnp.float32),
            pltpu.SemaphoreType.DMA((6,)),
            pltpu.SemaphoreType.DMA((4,)),
        ],
    )
    return kern(src0, dst0, src1, dst1, ee, feat0, feat1)


def _k2_body(mp_ref, sp_ref, b0_ref, b1_ref, w1_ref, b1s_ref, w2_ref,
             z0_ref, z1_ref, tw_ref):
    mp = mp_ref[...]
    sp = sp_ref[...]
    m0 = mp[0] + mp[2]
    m1 = mp[1] + mp[3]
    s0 = (jnp.sum(sp[:, 0:16], axis=1, keepdims=True)
          + jnp.sum(sp[:, 32:48], axis=1, keepdims=True))
    s1 = (jnp.sum(sp[:, 16:32], axis=1, keepdims=True)
          + jnp.sum(sp[:, 48:64], axis=1, keepdims=True))
    d = 128

    def path(m, s, b_ref):
        x = m / (s + 1e-9) + b_ref[...]
        return jnp.where(x > 0.0, x, jnp.exp(x) - 1.0)

    z0 = path(m0, s0, b0_ref)
    z1 = path(m1, s1, b1_ref)
    z0_ref[...] = z0
    z1_ref[...] = z1

    def sem(z):
        t = jnp.tanh(jnp.dot(z, w1_ref[...], preferred_element_type=jnp.float32,
                             precision=lax.Precision.HIGHEST) + b1s_ref[...])
        return jnp.sum(jnp.dot(t, w2_ref[...], preferred_element_type=jnp.float32,
                               precision=lax.Precision.HIGHEST))

    col = lax.broadcasted_iota(jnp.int32, (1, 1, d), 2)
    tw_ref[...] = (jnp.where(col == 0, sem(z0), 0.0)
                   + jnp.where(col == 1, sem(z1), 0.0))


def _k2_call(msgp, s2d, b0, b1, sem_W1, sem_b1, sem_W2):
    n = msgp.shape[1]
    d = 128
    hid = sem_W1.shape[1]
    rb = 1000
    grid = (n // rb,)
    full = lambda s: pl.BlockSpec(s, lambda i: (0, 0))
    return pl.pallas_call(
        _k2_body,
        grid=grid,
        in_specs=[
            pl.BlockSpec((4, rb, d), lambda i: (0, i, 0)),
            pl.BlockSpec((rb, 64), lambda i: (i, 0)),
            full((1, d)), full((1, d)),
            full((d, hid)), full((1, hid)), full((hid, 1)),
        ],
        out_specs=[
            pl.BlockSpec((rb, d), lambda i: (i, 0)),
            pl.BlockSpec((rb, d), lambda i: (i, 0)),
            pl.BlockSpec((1, 1, d), lambda i: (i, 0, 0)),
        ],
        out_shape=[
            jax.ShapeDtypeStruct((n, d), jnp.float32),
            jax.ShapeDtypeStruct((n, d), jnp.float32),
            jax.ShapeDtypeStruct((grid[0], 1, d), jnp.float32),
        ],
    )(msgp, s2d, b0.reshape(1, d), b1.reshape(1, d),
      sem_W1, sem_b1.reshape(1, hid), sem_W2)


def _k3_body(n, tw_ref, z0_ref, z1_ref, o_ref):
    tws = jnp.sum(tw_ref[...], axis=0) * (1.0 / n)
    w0 = tws[:, 0:1]
    w1 = tws[:, 1:2]
    m = jnp.maximum(w0, w1)
    e0 = jnp.exp(w0 - m)
    e1 = jnp.exp(w1 - m)
    o_ref[...] = (e0 * z0_ref[...] + e1 * z1_ref[...]) / (e0 + e1)


def _k3_call(tw, z0, z1):
    n, d = z0.shape
    rb = 1000
    grid = (n // rb,)
    return pl.pallas_call(
        functools.partial(_k3_body, n),
        grid=grid,
        in_specs=[
            pl.BlockSpec(tw.shape, lambda i: (0, 0, 0)),
            pl.BlockSpec((rb, d), lambda i: (i, 0)),
            pl.BlockSpec((rb, d), lambda i: (i, 0)),
        ],
        out_specs=pl.BlockSpec((rb, d), lambda i: (i, 0)),
        out_shape=jax.ShapeDtypeStruct((n, d), jnp.float32),
    )(tw, z0, z1)


def kernel(h, edge_index0, edge_index1, W0, al0, ar0, b0, W1, al1, ar1, b1,
           sem_W1, sem_b1, sem_W2):
    n, d = h.shape
    f0, f1, el0, er0, el1, er1 = _mm_call(h, W0, al0, ar0, W1, al1, ar1)
    s0a, d0a = edge_index0[0], edge_index0[1]
    s1a, d1a = edge_index1[0], edge_index1[1]
    ee, s_part = _sca_call(s0a, d0a, s1a, d1a,
                           el0.reshape(n), er0.reshape(n),
                           el1.reshape(n), er1.reshape(n))
    msgp = _scb_call(s0a, d0a, s1a, d1a, ee, f0, f1)
    z0, z1, tw = _k2_call(msgp, s_part.T, b0, b1, sem_W1, sem_b1, sem_W2)
    return _k3_call(tw, z0, z1)
